# baseline (speedup 1.0000x reference)
import math

import jax
import jax.numpy as jnp
from jax import lax
from jax.experimental import pallas as pl
from jax.experimental.pallas import tpu as pltpu

_TOP_K = 2
_M = 8
_D = 768
_B = 4
_T = 2048
_F = _T // 2 - 1
_FP = 1024
_ROWS = _B * _D
_RBLK = 384
_WPAD = _T + 128
import numpy as _np
_TWO_PI = _np.float32(2.0 * math.pi)
_HI = lax.Precision.HIGHEST


def _dft_tables():
    n = _np.arange(_T, dtype=_np.int64)[:, None]
    k = _np.arange(1, _F + 1, dtype=_np.int64)[None, :]
    phi = ((n * k) % _T).astype(_np.float32).astype(_np.float64) \
        * float(_np.float32(2.0 * math.pi / _T))
    cos = _np.cos(phi).astype(_np.float32)
    nsin = (-_np.sin(phi)).astype(_np.float32)
    pad = _np.zeros((_T, _FP - _F), _np.float32)
    return (_np.concatenate([cos, pad], axis=1),
            _np.concatenate([nsin, pad], axis=1))


_COS_TAB, _NSIN_TAB = _dft_tables()


def _shift(a, s):
    if s == 0:
        return a
    r = a.shape[0]
    return jnp.concatenate(
        [a[:, s:], jnp.zeros((r, s), jnp.float32)], axis=1)


def _pick(mask, v):
    return jnp.sum(jnp.where(mask, v, 0.0), axis=1, keepdims=True)


def _seasontrend_kernel(x_ref, cos_ref, nsin_ref, wt_ref, bt_ref, y_ref):
    x = x_ref[...]
    re = jnp.dot(x, cos_ref[...], preferred_element_type=jnp.float32,
                 precision=_HI)
    im = jnp.dot(x, nsin_ref[...], preferred_element_type=jnp.float32,
                 precision=_HI)
    mag2 = re * re + im * im
    iota = lax.broadcasted_iota(jnp.int32, mag2.shape, 1)
    big = jnp.int32(_FP)
    v1 = jnp.max(mag2, axis=1, keepdims=True)
    i1 = jnp.min(jnp.where(mag2 == v1, iota, big), axis=1, keepdims=True)
    m2 = jnp.where(iota == i1, jnp.float32(-1.0), mag2)
    v2 = jnp.max(m2, axis=1, keepdims=True)
    i2 = jnp.min(jnp.where(m2 == v2, iota, big), axis=1, keepdims=True)

    r = x.shape[0]
    tq = lax.broadcasted_iota(jnp.int32, (r, _T), 1).astype(jnp.float32)
    seas = jnp.zeros((r, _T), jnp.float32)
    for idx in (i1, i2):
        keep = iota == idx
        rek = _pick(keep, re)
        imk = _pick(keep, im)
        amp2 = jnp.sqrt(rek * rek + imk * imk) * jnp.float32(2.0 / _T)
        phase = jnp.arctan2(imk, rek)
        f = (idx + 1).astype(jnp.float32) * jnp.float32(1.0 / _T)
        w = _TWO_PI * f
        seas = seas + amp2 * jnp.cos(w * tq + phase)
    rem = x - seas

    front = jnp.broadcast_to(rem[:, 0:1], (r, 16))
    back = jnp.broadcast_to(rem[:, _T - 1:_T], (r, _WPAD - _T - 16))
    xp = jnp.concatenate([front, rem, back], axis=1)
    w4 = xp + _shift(xp, 1) + _shift(xp, 2) + _shift(xp, 3)
    w8 = w4 + _shift(w4, 4)
    w16 = w8 + _shift(w8, 8)
    w32 = w16 + _shift(w16, 16)
    mas = []
    for k, wk in ((4, w4), (8, w8), (16, w16), (32, w32)):
        fk = k - 1 - (k - 1) // 2
        mas.append(_shift(wk, 16 - fk)[:, :_T] * jnp.float32(1.0 / k))
    logits = [rem * wt_ref[0, j] + bt_ref[0, j] for j in range(4)]
    mx = jnp.maximum(jnp.maximum(logits[0], logits[1]),
                     jnp.maximum(logits[2], logits[3]))
    es = [jnp.exp(l - mx) for l in logits]
    tot = es[0] + es[1] + es[2] + es[3]
    trend = (mas[0] * es[0] + mas[1] * es[1]
             + mas[2] * es[2] + mas[3] * es[3]) / tot
    y_ref[...] = x + seas + trend


def _route_kernel(y_ref, wlin_ref, blin_ref, wr_ref, br_ref,
                  wn_ref, bn_ref, noise_ref, out_ref):
    y = y_ref[0].astype(jnp.bfloat16)
    xt = jnp.dot(wlin_ref[...].astype(jnp.bfloat16), y,
                 preferred_element_type=jnp.float32)
    xt = xt + blin_ref[...]
    xtb = xt.astype(jnp.bfloat16)
    base = jnp.dot(wr_ref[...].astype(jnp.bfloat16), xtb,
                   preferred_element_type=jnp.float32)
    base = base + br_ref[...]
    nb = jnp.dot(wn_ref[...].astype(jnp.bfloat16), xtb,
                 preferred_element_type=jnp.float32)
    nb = nb + bn_ref[...]
    sp = jnp.logaddexp(nb, jnp.float32(0.0))
    raw = base + noise_ref[0] * sp
    mx = jnp.max(raw, axis=0, keepdims=True)
    e = jnp.exp(raw - mx)
    pw = e / jnp.sum(e, axis=0, keepdims=True)
    iota = lax.broadcasted_iota(jnp.int32, pw.shape, 0)
    rank = jnp.zeros(pw.shape, jnp.float32)
    for j in range(_M):
        pj = pw[j:j + 1, :]
        gt = (pj > pw).astype(jnp.float32)
        tie = jnp.logical_and(pj == pw, iota > j).astype(jnp.float32)
        rank = rank + gt + tie
    out_ref[0] = jnp.where(rank < jnp.float32(_TOP_K), pw, 0.0)


def kernel(x, W_lin, b_lin, W_trend, b_trend, W_r, b_r, W_n, b_n, noise):
    xt = jnp.transpose(x[:, :, :, 0], (0, 2, 1)).reshape(_ROWS, _T)
    cos_t = jnp.asarray(_COS_TAB)
    nsin_t = jnp.asarray(_NSIN_TAB)

    nblk = _ROWS // _RBLK
    wt2 = W_trend[:, 0].reshape(1, 4)
    bt2 = b_trend.reshape(1, 4)
    y = pl.pallas_call(
        _seasontrend_kernel,
        grid=(nblk,),
        in_specs=[
            pl.BlockSpec((_RBLK, _T), lambda i: (i, 0)),
            pl.BlockSpec((_T, _FP), lambda i: (0, 0)),
            pl.BlockSpec((_T, _FP), lambda i: (0, 0)),
            pl.BlockSpec((1, 4), lambda i: (0, 0)),
            pl.BlockSpec((1, 4), lambda i: (0, 0)),
        ],
        out_specs=pl.BlockSpec((_RBLK, _T), lambda i: (i, 0)),
        out_shape=jax.ShapeDtypeStruct((_ROWS, _T), jnp.float32),
    )(xt, cos_t, nsin_t, wt2, bt2)

    noise_t = jnp.transpose(noise, (0, 2, 1))
    out_t = pl.pallas_call(
        _route_kernel,
        grid=(_B,),
        in_specs=[
            pl.BlockSpec((1, _D, _T), lambda b: (b, 0, 0)),
            pl.BlockSpec((_D, _D), lambda b: (0, 0)),
            pl.BlockSpec((_D, 1), lambda b: (0, 0)),
            pl.BlockSpec((_M, _D), lambda b: (0, 0)),
            pl.BlockSpec((_M, 1), lambda b: (0, 0)),
            pl.BlockSpec((_M, _D), lambda b: (0, 0)),
            pl.BlockSpec((_M, 1), lambda b: (0, 0)),
            pl.BlockSpec((1, _M, _T), lambda b: (b, 0, 0)),
        ],
        out_specs=pl.BlockSpec((1, _M, _T), lambda b: (b, 0, 0)),
        out_shape=jax.ShapeDtypeStruct((_B, _M, _T), jnp.float32),
    )(y.reshape(_B, _D, _T), W_lin, b_lin.reshape(_D, 1),
      W_r, b_r.reshape(_M, 1), W_n, b_n.reshape(_M, 1), noise_t)

    return jnp.transpose(out_t, (0, 2, 1))


# speedup vs baseline: 1.1785x; 1.1785x over previous
import math

import jax
import jax.numpy as jnp
from jax import lax
from jax.experimental import pallas as pl
from jax.experimental.pallas import tpu as pltpu

_TOP_K = 2
_M = 8
_D = 768
_B = 4
_T = 2048
_F = _T // 2 - 1
_FP = 1024
_ROWS = _B * _D
_RBLK = 256
_WPAD = _T + 128
import numpy as _np
_TWO_PI = _np.float32(2.0 * math.pi)
_HI = lax.Precision.HIGHEST


def _dft_tables():
    n = _np.arange(_T, dtype=_np.int64)[:, None]
    k = _np.arange(1, _F + 1, dtype=_np.int64)[None, :]
    phi = ((n * k) % _T).astype(_np.float32).astype(_np.float64) \
        * float(_np.float32(2.0 * math.pi / _T))
    cos = _np.cos(phi).astype(_np.float32)
    nsin = (-_np.sin(phi)).astype(_np.float32)
    pad = _np.zeros((_T, _FP - _F), _np.float32)
    return (_np.concatenate([cos, pad], axis=1),
            _np.concatenate([nsin, pad], axis=1))


_COS_TAB, _NSIN_TAB = _dft_tables()


def _shift(a, s):
    if s == 0:
        return a
    r = a.shape[0]
    return jnp.concatenate(
        [a[:, s:], jnp.zeros((r, s), jnp.float32)], axis=1)


def _pick(mask, v):
    return jnp.sum(jnp.where(mask, v, 0.0), axis=1, keepdims=True)


def _seasontrend_kernel(x_ref, cos_ref, nsin_ref, wt_ref, bt_ref, y_ref):
    x = x_ref[...]
    re = jnp.dot(x, cos_ref[...], preferred_element_type=jnp.float32,
                 precision=_HI)
    im = jnp.dot(x, nsin_ref[...], preferred_element_type=jnp.float32,
                 precision=_HI)
    mag2 = re * re + im * im
    iota = lax.broadcasted_iota(jnp.int32, mag2.shape, 1)
    big = jnp.int32(_FP)
    v1 = jnp.max(mag2, axis=1, keepdims=True)
    i1 = jnp.min(jnp.where(mag2 == v1, iota, big), axis=1, keepdims=True)
    m2 = jnp.where(iota == i1, jnp.float32(-1.0), mag2)
    v2 = jnp.max(m2, axis=1, keepdims=True)
    i2 = jnp.min(jnp.where(m2 == v2, iota, big), axis=1, keepdims=True)

    r = x.shape[0]
    tq = lax.broadcasted_iota(jnp.int32, (r, _T), 1).astype(jnp.float32)
    seas = jnp.zeros((r, _T), jnp.float32)
    for idx in (i1, i2):
        keep = iota == idx
        rek = _pick(keep, re)
        imk = _pick(keep, im)
        amp2 = jnp.sqrt(rek * rek + imk * imk) * jnp.float32(2.0 / _T)
        phase = jnp.arctan2(imk, rek)
        f = (idx + 1).astype(jnp.float32) * jnp.float32(1.0 / _T)
        w = _TWO_PI * f
        seas = seas + amp2 * jnp.cos(w * tq + phase)
    rem = x - seas

    front = jnp.broadcast_to(rem[:, 0:1], (r, 16))
    back = jnp.broadcast_to(rem[:, _T - 1:_T], (r, _WPAD - _T - 16))
    xp = jnp.concatenate([front, rem, back], axis=1)
    w4 = xp + _shift(xp, 1) + _shift(xp, 2) + _shift(xp, 3)
    w8 = w4 + _shift(w4, 4)
    w16 = w8 + _shift(w8, 8)
    w32 = w16 + _shift(w16, 16)
    mas = []
    for k, wk in ((4, w4), (8, w8), (16, w16), (32, w32)):
        fk = k - 1 - (k - 1) // 2
        mas.append(_shift(wk, 16 - fk)[:, :_T] * jnp.float32(1.0 / k))
    logits = [rem * wt_ref[0, j] + bt_ref[0, j] for j in range(4)]
    mx = jnp.maximum(jnp.maximum(logits[0], logits[1]),
                     jnp.maximum(logits[2], logits[3]))
    es = [jnp.exp(l - mx) for l in logits]
    tot = es[0] + es[1] + es[2] + es[3]
    trend = (mas[0] * es[0] + mas[1] * es[1]
             + mas[2] * es[2] + mas[3] * es[3]) / tot
    y_ref[...] = x + seas + trend


def _route_kernel(y_ref, wlin_ref, blin_ref, wr_ref, br_ref,
                  wn_ref, bn_ref, noise_ref, out_ref):
    y = y_ref[0].astype(jnp.bfloat16)
    xt = jnp.dot(wlin_ref[...].astype(jnp.bfloat16), y,
                 preferred_element_type=jnp.float32)
    xt = xt + blin_ref[...]
    xtb = xt.astype(jnp.bfloat16)
    base = jnp.dot(wr_ref[...].astype(jnp.bfloat16), xtb,
                   preferred_element_type=jnp.float32)
    base = base + br_ref[...]
    nb = jnp.dot(wn_ref[...].astype(jnp.bfloat16), xtb,
                 preferred_element_type=jnp.float32)
    nb = nb + bn_ref[...]
    sp = jnp.logaddexp(nb, jnp.float32(0.0))
    raw = base + noise_ref[0] * sp
    mx = jnp.max(raw, axis=0, keepdims=True)
    e = jnp.exp(raw - mx)
    pw = e / jnp.sum(e, axis=0, keepdims=True)
    iota = lax.broadcasted_iota(jnp.int32, pw.shape, 0)
    rank = jnp.zeros(pw.shape, jnp.float32)
    for j in range(_M):
        pj = pw[j:j + 1, :]
        gt = (pj > pw).astype(jnp.float32)
        tie = jnp.logical_and(pj == pw, iota > j).astype(jnp.float32)
        rank = rank + gt + tie
    out_ref[0] = jnp.where(rank < jnp.float32(_TOP_K), pw, 0.0)


def kernel(x, W_lin, b_lin, W_trend, b_trend, W_r, b_r, W_n, b_n, noise):
    xt = jnp.transpose(x[:, :, :, 0], (0, 2, 1)).reshape(_ROWS, _T)
    cos_t = jnp.asarray(_COS_TAB)
    nsin_t = jnp.asarray(_NSIN_TAB)

    nblk = _ROWS // _RBLK
    wt2 = W_trend[:, 0].reshape(1, 4)
    bt2 = b_trend.reshape(1, 4)
    y = pl.pallas_call(
        _seasontrend_kernel,
        grid=(nblk,),
        in_specs=[
            pl.BlockSpec((_RBLK, _T), lambda i: (i, 0)),
            pl.BlockSpec((_T, _FP), lambda i: (0, 0)),
            pl.BlockSpec((_T, _FP), lambda i: (0, 0)),
            pl.BlockSpec((1, 4), lambda i: (0, 0)),
            pl.BlockSpec((1, 4), lambda i: (0, 0)),
        ],
        out_specs=pl.BlockSpec((_RBLK, _T), lambda i: (i, 0)),
        out_shape=jax.ShapeDtypeStruct((_ROWS, _T), jnp.float32),
    )(xt, cos_t, nsin_t, wt2, bt2)

    noise_t = jnp.transpose(noise, (0, 2, 1))
    out_t = pl.pallas_call(
        _route_kernel,
        grid=(_B,),
        in_specs=[
            pl.BlockSpec((1, _D, _T), lambda b: (b, 0, 0)),
            pl.BlockSpec((_D, _D), lambda b: (0, 0)),
            pl.BlockSpec((_D, 1), lambda b: (0, 0)),
            pl.BlockSpec((_M, _D), lambda b: (0, 0)),
            pl.BlockSpec((_M, 1), lambda b: (0, 0)),
            pl.BlockSpec((_M, _D), lambda b: (0, 0)),
            pl.BlockSpec((_M, 1), lambda b: (0, 0)),
            pl.BlockSpec((1, _M, _T), lambda b: (b, 0, 0)),
        ],
        out_specs=pl.BlockSpec((1, _M, _T), lambda b: (b, 0, 0)),
        out_shape=jax.ShapeDtypeStruct((_B, _M, _T), jnp.float32),
    )(y.reshape(_B, _D, _T), W_lin, b_lin.reshape(_D, 1),
      W_r, b_r.reshape(_M, 1), W_n, b_n.reshape(_M, 1), noise_t)

    return jnp.transpose(out_t, (0, 2, 1))


# speedup vs baseline: 1.4281x; 1.2118x over previous
import math

import jax
import jax.numpy as jnp
from jax import lax
from jax.experimental import pallas as pl
from jax.experimental.pallas import tpu as pltpu

_TOP_K = 2
_M = 8
_D = 768
_B = 4
_T = 2048
_F = _T // 2 - 1
_FP = 1024
_ROWS = _B * _D
_RBLK = 256
_WPAD = _T + 128
import numpy as _np
_TWO_PI = _np.float32(2.0 * math.pi)
_HI = lax.Precision.HIGH


def _dft_tables():
    n = _np.arange(_T, dtype=_np.int64)[:, None]
    k = _np.arange(1, _F + 1, dtype=_np.int64)[None, :]
    phi = ((n * k) % _T).astype(_np.float32).astype(_np.float64) \
        * float(_np.float32(2.0 * math.pi / _T))
    cos = _np.cos(phi).astype(_np.float32)
    nsin = (-_np.sin(phi)).astype(_np.float32)
    pad = _np.zeros((_T, _FP - _F), _np.float32)
    return (_np.concatenate([cos, pad], axis=1),
            _np.concatenate([nsin, pad], axis=1))


_COS_TAB, _NSIN_TAB = _dft_tables()


def _hi_lo(a):
    import ml_dtypes
    hi = a.astype(ml_dtypes.bfloat16)
    lo = (a - hi.astype(_np.float32)).astype(ml_dtypes.bfloat16)
    return hi, lo


_COS_HI, _COS_LO = _hi_lo(_COS_TAB)
_NSIN_HI, _NSIN_LO = _hi_lo(_NSIN_TAB)


def _shift(a, s):
    if s == 0:
        return a
    r = a.shape[0]
    return jnp.concatenate(
        [a[:, s:], jnp.zeros((r, s), jnp.float32)], axis=1)


def _pick(mask, v):
    return jnp.sum(jnp.where(mask, v, 0.0), axis=1, keepdims=True)


def _seasontrend_kernel(x_ref, cos_hi_ref, cos_lo_ref, nsin_hi_ref,
                        nsin_lo_ref, wt_ref, bt_ref, y_ref):
    x = x_ref[...]
    x_hi = x.astype(jnp.bfloat16)
    x_lo = (x - x_hi.astype(jnp.float32)).astype(jnp.bfloat16)
    ch = cos_hi_ref[...]
    re = (jnp.dot(x_hi, ch, preferred_element_type=jnp.float32)
          + (jnp.dot(x_hi, cos_lo_ref[...],
                     preferred_element_type=jnp.float32)
             + jnp.dot(x_lo, ch, preferred_element_type=jnp.float32)))
    sh = nsin_hi_ref[...]
    im = (jnp.dot(x_hi, sh, preferred_element_type=jnp.float32)
          + (jnp.dot(x_hi, nsin_lo_ref[...],
                     preferred_element_type=jnp.float32)
             + jnp.dot(x_lo, sh, preferred_element_type=jnp.float32)))
    mag2 = re * re + im * im
    iota = lax.broadcasted_iota(jnp.int32, mag2.shape, 1)
    big = jnp.int32(_FP)
    v1 = jnp.max(mag2, axis=1, keepdims=True)
    i1 = jnp.min(jnp.where(mag2 == v1, iota, big), axis=1, keepdims=True)
    m2 = jnp.where(iota == i1, jnp.float32(-1.0), mag2)
    v2 = jnp.max(m2, axis=1, keepdims=True)
    i2 = jnp.min(jnp.where(m2 == v2, iota, big), axis=1, keepdims=True)

    r = x.shape[0]
    tq = lax.broadcasted_iota(jnp.int32, (r, _T), 1).astype(jnp.float32)
    seas = jnp.zeros((r, _T), jnp.float32)
    for idx in (i1, i2):
        keep = iota == idx
        rek = _pick(keep, re)
        imk = _pick(keep, im)
        amp2 = jnp.sqrt(rek * rek + imk * imk) * jnp.float32(2.0 / _T)
        phase = jnp.arctan2(imk, rek)
        f = (idx + 1).astype(jnp.float32) * jnp.float32(1.0 / _T)
        w = _TWO_PI * f
        seas = seas + amp2 * jnp.cos(w * tq + phase)
    rem = x - seas

    front = jnp.broadcast_to(rem[:, 0:1], (r, 16))
    back = jnp.broadcast_to(rem[:, _T - 1:_T], (r, _WPAD - _T - 16))
    xp = jnp.concatenate([front, rem, back], axis=1)
    w4 = xp + _shift(xp, 1) + _shift(xp, 2) + _shift(xp, 3)
    w8 = w4 + _shift(w4, 4)
    w16 = w8 + _shift(w8, 8)
    w32 = w16 + _shift(w16, 16)
    mas = []
    for k, wk in ((4, w4), (8, w8), (16, w16), (32, w32)):
        fk = k - 1 - (k - 1) // 2
        mas.append(_shift(wk, 16 - fk)[:, :_T] * jnp.float32(1.0 / k))
    logits = [rem * wt_ref[0, j] + bt_ref[0, j] for j in range(4)]
    mx = jnp.maximum(jnp.maximum(logits[0], logits[1]),
                     jnp.maximum(logits[2], logits[3]))
    es = [jnp.exp(l - mx) for l in logits]
    tot = es[0] + es[1] + es[2] + es[3]
    trend = (mas[0] * es[0] + mas[1] * es[1]
             + mas[2] * es[2] + mas[3] * es[3]) / tot
    y_ref[...] = x + seas + trend


def _route_kernel(y_ref, wlin_ref, blin_ref, wr_ref, br_ref,
                  wn_ref, bn_ref, noise_ref, out_ref):
    y = y_ref[0].astype(jnp.bfloat16)
    xt = jnp.dot(wlin_ref[...].astype(jnp.bfloat16), y,
                 preferred_element_type=jnp.float32)
    xt = xt + blin_ref[...]
    xtb = xt.astype(jnp.bfloat16)
    base = jnp.dot(wr_ref[...].astype(jnp.bfloat16), xtb,
                   preferred_element_type=jnp.float32)
    base = base + br_ref[...]
    nb = jnp.dot(wn_ref[...].astype(jnp.bfloat16), xtb,
                 preferred_element_type=jnp.float32)
    nb = nb + bn_ref[...]
    sp = jnp.logaddexp(nb, jnp.float32(0.0))
    raw = base + noise_ref[0] * sp
    mx = jnp.max(raw, axis=0, keepdims=True)
    e = jnp.exp(raw - mx)
    pw = e / jnp.sum(e, axis=0, keepdims=True)
    iota = lax.broadcasted_iota(jnp.int32, pw.shape, 0)
    rank = jnp.zeros(pw.shape, jnp.float32)
    for j in range(_M):
        pj = pw[j:j + 1, :]
        gt = (pj > pw).astype(jnp.float32)
        tie = jnp.logical_and(pj == pw, iota > j).astype(jnp.float32)
        rank = rank + gt + tie
    out_ref[0] = jnp.where(rank < jnp.float32(_TOP_K), pw, 0.0)


def kernel(x, W_lin, b_lin, W_trend, b_trend, W_r, b_r, W_n, b_n, noise):
    xt = jnp.transpose(x[:, :, :, 0], (0, 2, 1)).reshape(_ROWS, _T)

    nblk = _ROWS // _RBLK
    wt2 = W_trend[:, 0].reshape(1, 4)
    bt2 = b_trend.reshape(1, 4)
    tab_spec = pl.BlockSpec((_T, _FP), lambda i: (0, 0))
    y = pl.pallas_call(
        _seasontrend_kernel,
        grid=(nblk,),
        in_specs=[
            pl.BlockSpec((_RBLK, _T), lambda i: (i, 0)),
            tab_spec, tab_spec, tab_spec, tab_spec,
            pl.BlockSpec((1, 4), lambda i: (0, 0)),
            pl.BlockSpec((1, 4), lambda i: (0, 0)),
        ],
        out_specs=pl.BlockSpec((_RBLK, _T), lambda i: (i, 0)),
        out_shape=jax.ShapeDtypeStruct((_ROWS, _T), jnp.float32),
    )(xt, jnp.asarray(_COS_HI), jnp.asarray(_COS_LO),
      jnp.asarray(_NSIN_HI), jnp.asarray(_NSIN_LO), wt2, bt2)

    noise_t = jnp.transpose(noise, (0, 2, 1))
    out_t = pl.pallas_call(
        _route_kernel,
        grid=(_B,),
        in_specs=[
            pl.BlockSpec((1, _D, _T), lambda b: (b, 0, 0)),
            pl.BlockSpec((_D, _D), lambda b: (0, 0)),
            pl.BlockSpec((_D, 1), lambda b: (0, 0)),
            pl.BlockSpec((_M, _D), lambda b: (0, 0)),
            pl.BlockSpec((_M, 1), lambda b: (0, 0)),
            pl.BlockSpec((_M, _D), lambda b: (0, 0)),
            pl.BlockSpec((_M, 1), lambda b: (0, 0)),
            pl.BlockSpec((1, _M, _T), lambda b: (b, 0, 0)),
        ],
        out_specs=pl.BlockSpec((1, _M, _T), lambda b: (b, 0, 0)),
        out_shape=jax.ShapeDtypeStruct((_B, _M, _T), jnp.float32),
    )(y.reshape(_B, _D, _T), W_lin, b_lin.reshape(_D, 1),
      W_r, b_r.reshape(_M, 1), W_n, b_n.reshape(_M, 1), noise_t)

    return jnp.transpose(out_t, (0, 2, 1))
